# initial kernel scaffold (unmeasured)
import jax
import jax.numpy as jnp
from jax import lax
from jax.experimental import pallas as pl
from jax.experimental.pallas import tpu as pltpu

N_DEV = 16
N_ROUNDS = 4
SCALE = 0.08838834764831843
QBLK = 64
N_QB = 4


def kernel(x, Wq, K_ext, V_ext, Wo):
    B, Sq, Dm = x.shape
    _, Skv_l, Hq, Dh = K_ext.shape

    def body(x_ref, wq_ref, k_ref, v_ref, wo_ref, out_ref,
             ctx_send, ctx_recv, st_send, st_recv,
             ctx_ssem, ctx_rsem, st_ssem, st_rsem):
        my = lax.axis_index("i")

        Q = jnp.dot(x_ref[0], wq_ref[...],
                    preferred_element_type=jnp.float32)
        Kv = k_ref[0]
        Vv = v_ref[0]
        n_grp = Skv_l // (4 * QBLK)
        Kr = Kv.reshape(n_grp, 4, QBLK, Hq, Dh)
        Vr = Vv.reshape(n_grp, 4, QBLK, Hq, Dh)

        m_parts, l_parts, ctx_parts = [], [], []
        for qb in range(N_QB):
            Ksub = Kr[:, qb].reshape(n_grp * QBLK, Hq, Dh)
            Vsub = Vr[:, qb].reshape(n_grp * QBLK, Hq, Dh)
            Qb = Q[qb * QBLK:(qb + 1) * QBLK].reshape(QBLK, Hq, Dh)
            m_h, l_h, c_h = [], [], []
            for h in range(Hq):
                qh = Qb[:, h, :]
                kh = Ksub[:, h, :]
                vh = Vsub[:, h, :]
                s = lax.dot_general(
                    qh, kh, (((1,), (1,)), ((), ())),
                    preferred_element_type=jnp.float32) * SCALE
                m = jnp.max(s, axis=1)
                w = jnp.exp(s - m[:, None])
                l = jnp.sum(w, axis=1)
                c = jnp.dot(w, vh,
                            preferred_element_type=jnp.float32)
                m_h.append(m)
                l_h.append(l)
                c_h.append(c)
            m_parts.append(jnp.stack(m_h))
            l_parts.append(jnp.stack(l_h))
            ctx_parts.append(jnp.stack(c_h))

        ctx_send[...] = jnp.concatenate(ctx_parts, axis=1)
        st_send[0] = jnp.concatenate(m_parts, axis=1)
        st_send[1] = jnp.concatenate(l_parts, axis=1)

        for r in range(N_ROUNDS):
            partner = my ^ (1 << r)
            c_rdma = pltpu.make_async_remote_copy(
                src_ref=ctx_send, dst_ref=ctx_recv.at[r],
                send_sem=ctx_ssem.at[r], recv_sem=ctx_rsem.at[r],
                device_id=(partner,), device_id_type=pl.DeviceIdType.MESH)
            s_rdma = pltpu.make_async_remote_copy(
                src_ref=st_send, dst_ref=st_recv.at[r],
                send_sem=st_ssem.at[r], recv_sem=st_rsem.at[r],
                device_id=(partner,), device_id_type=pl.DeviceIdType.MESH)
            c_rdma.start()
            s_rdma.start()
            c_rdma.wait()
            s_rdma.wait()

            m_c = st_send[0]
            l_c = st_send[1]
            ctx_c = ctx_send[...]
            m_p = st_recv[r, 0]
            l_p = st_recv[r, 1]
            ctx_p = ctx_recv[r]
            m_n = jnp.maximum(m_c, m_p)
            a = jnp.exp(m_c - m_n)
            b = jnp.exp(m_p - m_n)
            ctx_n = ctx_c * a[:, :, None] + ctx_p * b[:, :, None]
            l_n = l_c * a + l_p * b
            if r < N_ROUNDS - 1:
                ctx_send[...] = ctx_n
                st_send[0] = m_n
                st_send[1] = l_n
            else:
                ctx_f = ctx_n / l_n[:, :, None]
                ctx2 = jnp.swapaxes(ctx_f, 0, 1).reshape(Sq, Hq * Dh)
                out_ref[0] = jnp.dot(ctx2, wo_ref[...],
                                     preferred_element_type=jnp.float32)

    return pl.pallas_call(
        body,
        out_shape=jax.ShapeDtypeStruct((B, Sq, Dm), jnp.float32),
        in_specs=[pl.BlockSpec(memory_space=pltpu.VMEM)] * 5,
        out_specs=pl.BlockSpec(memory_space=pltpu.VMEM),
        scratch_shapes=[
            pltpu.VMEM((Hq, Sq, Dh), jnp.float32),
            pltpu.VMEM((N_ROUNDS, Hq, Sq, Dh), jnp.float32),
            pltpu.VMEM((2, Hq, Sq), jnp.float32),
            pltpu.VMEM((N_ROUNDS, 2, Hq, Sq), jnp.float32),
            pltpu.SemaphoreType.DMA((N_ROUNDS,)),
            pltpu.SemaphoreType.DMA((N_ROUNDS,)),
            pltpu.SemaphoreType.DMA((N_ROUNDS,)),
            pltpu.SemaphoreType.DMA((N_ROUNDS,)),
        ],
        compiler_params=pltpu.CompilerParams(collective_id=0),
    )(x, Wq, K_ext, V_ext, Wo)


# baseline (device time: 116416 ns/iter reference)
import jax
import jax.numpy as jnp
from jax import lax
from jax.experimental import pallas as pl
from jax.experimental.pallas import tpu as pltpu

N_DEV = 16
N_ROUNDS = 4
SCALE = 0.08838834764831843
QBLK = 64
N_QB = 4


def kernel(x, Wq, K_ext, V_ext, Wo):
    B, Sq, Dm = x.shape
    _, Skv_l, Hq, Dh = K_ext.shape

    def body(x_ref, wq_ref, k_ref, v_ref, wo_ref, out_ref,
             ctx_send, ctx_recv, st_send, st_recv,
             ctx_ssem, ctx_rsem, st_ssem, st_rsem):
        my = lax.axis_index("i")

        Q = jnp.dot(x_ref[0], wq_ref[...],
                    preferred_element_type=jnp.float32)
        Kv = k_ref[0]
        Vv = v_ref[0]
        n_grp = Skv_l // (4 * QBLK)
        Kr = Kv.reshape(n_grp, 4, QBLK, Hq, Dh)
        Vr = Vv.reshape(n_grp, 4, QBLK, Hq, Dh)

        m_parts, l_parts, ctx_parts = [], [], []
        for qb in range(N_QB):
            Ksub = Kr[:, qb].reshape(n_grp * QBLK, Hq, Dh)
            Vsub = Vr[:, qb].reshape(n_grp * QBLK, Hq, Dh)
            Qb = Q[qb * QBLK:(qb + 1) * QBLK].reshape(QBLK, Hq, Dh)
            m_h, l_h, c_h = [], [], []
            for h in range(Hq):
                qh = Qb[:, h, :]
                kh = Ksub[:, h, :]
                vh = Vsub[:, h, :]
                s = lax.dot_general(
                    qh, kh, (((1,), (1,)), ((), ())),
                    preferred_element_type=jnp.float32) * SCALE
                m = jnp.max(s, axis=1)
                w = jnp.exp(s - m[:, None])
                l = jnp.sum(w, axis=1)
                c = jnp.dot(w, vh,
                            preferred_element_type=jnp.float32)
                m_h.append(m)
                l_h.append(l)
                c_h.append(c)
            m_parts.append(jnp.stack(m_h))
            l_parts.append(jnp.stack(l_h))
            ctx_parts.append(jnp.stack(c_h))

        ctx_send[...] = jnp.concatenate(ctx_parts, axis=1)
        st_send[0] = jnp.concatenate(m_parts, axis=1)
        st_send[1] = jnp.concatenate(l_parts, axis=1)

        for r in range(N_ROUNDS):
            partner = my ^ (1 << r)
            c_rdma = pltpu.make_async_remote_copy(
                src_ref=ctx_send, dst_ref=ctx_recv.at[r],
                send_sem=ctx_ssem.at[r], recv_sem=ctx_rsem.at[r],
                device_id=(partner,), device_id_type=pl.DeviceIdType.MESH)
            s_rdma = pltpu.make_async_remote_copy(
                src_ref=st_send, dst_ref=st_recv.at[r],
                send_sem=st_ssem.at[r], recv_sem=st_rsem.at[r],
                device_id=(partner,), device_id_type=pl.DeviceIdType.MESH)
            c_rdma.start()
            s_rdma.start()
            c_rdma.wait()
            s_rdma.wait()

            m_c = st_send[0]
            l_c = st_send[1]
            ctx_c = ctx_send[...]
            m_p = st_recv[r, 0]
            l_p = st_recv[r, 1]
            ctx_p = ctx_recv[r]
            m_n = jnp.maximum(m_c, m_p)
            a = jnp.exp(m_c - m_n)
            b = jnp.exp(m_p - m_n)
            ctx_n = ctx_c * a[:, :, None] + ctx_p * b[:, :, None]
            l_n = l_c * a + l_p * b
            if r < N_ROUNDS - 1:
                ctx_send[...] = ctx_n
                st_send[0] = m_n
                st_send[1] = l_n
            else:
                ctx_f = ctx_n / l_n[:, :, None]
                ctx2 = jnp.swapaxes(ctx_f, 0, 1).reshape(Sq, Hq * Dh)
                out_ref[0] = jnp.dot(ctx2, wo_ref[...],
                                     preferred_element_type=jnp.float32)

    return pl.pallas_call(
        body,
        out_shape=jax.ShapeDtypeStruct((B, Sq, Dm), jnp.float32),
        in_specs=[pl.BlockSpec(memory_space=pltpu.VMEM)] * 5,
        out_specs=pl.BlockSpec(memory_space=pltpu.VMEM),
        scratch_shapes=[
            pltpu.VMEM((Hq, Sq, Dh), jnp.float32),
            pltpu.VMEM((N_ROUNDS, Hq, Sq, Dh), jnp.float32),
            pltpu.VMEM((2, Hq, Sq), jnp.float32),
            pltpu.VMEM((N_ROUNDS, 2, Hq, Sq), jnp.float32),
            pltpu.SemaphoreType.DMA((N_ROUNDS,)),
            pltpu.SemaphoreType.DMA((N_ROUNDS,)),
            pltpu.SemaphoreType.DMA((N_ROUNDS,)),
            pltpu.SemaphoreType.DMA((N_ROUNDS,)),
        ],
        compiler_params=pltpu.CompilerParams(
            vmem_limit_bytes=100 * 1024 * 1024,
        ),
    )(x, Wq, K_ext, V_ext, Wo)


# device time: 90930 ns/iter; 1.2803x vs baseline; 1.2803x over previous
import jax
import jax.numpy as jnp
from jax import lax
from jax.experimental import pallas as pl
from jax.experimental.pallas import tpu as pltpu

N_DEV = 16
N_ROUNDS = 4
SCALE = 0.08838834764831843
QBLK = 64
N_QB = 4


def kernel(x, Wq, K_ext, V_ext, Wo):
    B, Sq, Dm = x.shape
    _, Skv_l, Hq, Dh = K_ext.shape

    def body(x_ref, wq_ref, k_ref, v_ref, wo_ref, out_ref,
             ctx_send, ctx_recv, st_send, st_recv,
             ctx_ssem, ctx_rsem, st_ssem, st_rsem):
        my = lax.axis_index("i")

        Q = jnp.dot(x_ref[0].astype(jnp.bfloat16),
                    wq_ref[...].astype(jnp.bfloat16),
                    preferred_element_type=jnp.float32)
        Kv = k_ref[0].astype(jnp.bfloat16)
        Vv = v_ref[0].astype(jnp.bfloat16)
        n_grp = Skv_l // (4 * QBLK)
        Kr = Kv.reshape(n_grp, 4, QBLK, Hq, Dh)
        Vr = Vv.reshape(n_grp, 4, QBLK, Hq, Dh)

        m_parts, l_parts, ctx_parts = [], [], []
        for qb in range(N_QB):
            Ksub = Kr[:, qb].reshape(n_grp * QBLK, Hq, Dh)
            Vsub = Vr[:, qb].reshape(n_grp * QBLK, Hq, Dh)
            Qb = Q[qb * QBLK:(qb + 1) * QBLK].reshape(
                QBLK, Hq, Dh).astype(jnp.bfloat16)
            m_h, l_h, c_h = [], [], []
            for h in range(Hq):
                qh = Qb[:, h, :]
                kh = Ksub[:, h, :]
                vh = Vsub[:, h, :]
                s = lax.dot_general(
                    qh, kh, (((1,), (1,)), ((), ())),
                    preferred_element_type=jnp.float32) * SCALE
                m = jnp.max(s, axis=1)
                w = jnp.exp(s - m[:, None])
                l = jnp.sum(w, axis=1)
                c = jnp.dot(w.astype(jnp.bfloat16), vh,
                            preferred_element_type=jnp.float32)
                m_h.append(m)
                l_h.append(l)
                c_h.append(c)
            m_parts.append(jnp.stack(m_h))
            l_parts.append(jnp.stack(l_h))
            ctx_parts.append(jnp.stack(c_h))

        ctx_send[...] = jnp.concatenate(
            ctx_parts, axis=1).astype(jnp.bfloat16)
        st_send[0] = jnp.concatenate(m_parts, axis=1)
        st_send[1] = jnp.concatenate(l_parts, axis=1)

        for r in range(N_ROUNDS):
            partner = my ^ (1 << r)
            c_rdma = pltpu.make_async_remote_copy(
                src_ref=ctx_send, dst_ref=ctx_recv.at[r],
                send_sem=ctx_ssem.at[r], recv_sem=ctx_rsem.at[r],
                device_id=(partner,), device_id_type=pl.DeviceIdType.MESH)
            s_rdma = pltpu.make_async_remote_copy(
                src_ref=st_send, dst_ref=st_recv.at[r],
                send_sem=st_ssem.at[r], recv_sem=st_rsem.at[r],
                device_id=(partner,), device_id_type=pl.DeviceIdType.MESH)
            c_rdma.start()
            s_rdma.start()
            c_rdma.wait()
            s_rdma.wait()

            m_c = st_send[0]
            l_c = st_send[1]
            ctx_c = ctx_send[...].astype(jnp.float32)
            m_p = st_recv[r, 0]
            l_p = st_recv[r, 1]
            ctx_p = ctx_recv[r].astype(jnp.float32)
            m_n = jnp.maximum(m_c, m_p)
            a = jnp.exp(m_c - m_n)
            b = jnp.exp(m_p - m_n)
            ctx_n = ctx_c * a[:, :, None] + ctx_p * b[:, :, None]
            l_n = l_c * a + l_p * b
            if r < N_ROUNDS - 1:
                ctx_send[...] = ctx_n.astype(jnp.bfloat16)
                st_send[0] = m_n
                st_send[1] = l_n
            else:
                ctx_f = ctx_n / l_n[:, :, None]
                ctx2 = jnp.swapaxes(ctx_f, 0, 1).reshape(Sq, Hq * Dh)
                out_ref[0] = jnp.dot(ctx2.astype(jnp.bfloat16),
                                     wo_ref[...].astype(jnp.bfloat16),
                                     preferred_element_type=jnp.float32)

    return pl.pallas_call(
        body,
        out_shape=jax.ShapeDtypeStruct((B, Sq, Dm), jnp.float32),
        in_specs=[pl.BlockSpec(memory_space=pltpu.VMEM)] * 5,
        out_specs=pl.BlockSpec(memory_space=pltpu.VMEM),
        scratch_shapes=[
            pltpu.VMEM((Hq, Sq, Dh), jnp.bfloat16),
            pltpu.VMEM((N_ROUNDS, Hq, Sq, Dh), jnp.bfloat16),
            pltpu.VMEM((2, Hq, Sq), jnp.float32),
            pltpu.VMEM((N_ROUNDS, 2, Hq, Sq), jnp.float32),
            pltpu.SemaphoreType.DMA((N_ROUNDS,)),
            pltpu.SemaphoreType.DMA((N_ROUNDS,)),
            pltpu.SemaphoreType.DMA((N_ROUNDS,)),
            pltpu.SemaphoreType.DMA((N_ROUNDS,)),
        ],
        compiler_params=pltpu.CompilerParams(
            vmem_limit_bytes=100 * 1024 * 1024,
        ),
    )(x, Wq, K_ext, V_ext, Wo)
